# baseline (speedup 1.0000x reference)
import functools

import jax
import jax.numpy as jnp
from jax.experimental import pallas as pl

LINK_R = 0.5


def _ceil_to(v, m):
    return (v + m - 1) // m * m


def _pad_layers(layer_params, k0_pad):
    out = []
    k_pad = k0_pad
    for (W, b) in layer_params:
        n_out, n_in = W.shape
        n_pad = _ceil_to(n_out, 128)
        Wp = jnp.zeros((k_pad, n_pad), jnp.float32).at[:n_in, :n_out].set(W.T)
        bp = jnp.zeros((1, n_pad), jnp.float32).at[0, :n_out].set(b)
        out.append((Wp, bp))
        k_pad = n_pad
    return out


def _mlp_body(n_layers, *refs):
    x_ref = refs[0]
    o_ref = refs[-1]
    h = x_ref[...]
    for i in range(n_layers):
        w = refs[1 + 2 * i][...]
        b = refs[2 + 2 * i][...]
        h = jnp.dot(h, w, preferred_element_type=jnp.float32) + b
        if i < n_layers - 1:
            h = jnp.maximum(h, 0.0)
    o_ref[...] = h


def _mlp_pallas(E, layers, bm):
    M, K = E.shape
    bm = min(bm, _ceil_to(M, 8))
    Mp = _ceil_to(M, bm)
    if Mp != M:
        E = jnp.pad(E, ((0, Mp - M), (0, 0)))
    n_layers = len(layers)
    n_out = layers[-1][0].shape[1]
    in_specs = [pl.BlockSpec((bm, K), lambda i: (i, 0))]
    for (W, b) in layers:
        in_specs.append(pl.BlockSpec(W.shape, lambda i: (0, 0)))
        in_specs.append(pl.BlockSpec(b.shape, lambda i: (0, 0)))
    flat = [a for Wb in layers for a in Wb]
    out = pl.pallas_call(
        functools.partial(_mlp_body, n_layers),
        grid=(Mp // bm,),
        in_specs=in_specs,
        out_specs=pl.BlockSpec((bm, n_out), lambda i: (i, 0)),
        out_shape=jax.ShapeDtypeStruct((Mp, n_out), jnp.float32),
    )(E, *flat)
    return out[:M] if Mp != M else out


def _segment_max0(data, ids, n):
    m = jax.ops.segment_max(data, ids, num_segments=n)
    return jnp.where(jnp.isneginf(m), 0.0, m)


def kernel(x, u, params, edge_index, batch):
    row, col = edge_index[0], edge_index[1]
    n = x.shape[0]
    n_graphs = u.shape[0]

    for lp in params["layers"]:
        ea = x[row, :3] - x[col, :3]
        ea = ea - (ea > LINK_R).astype(x.dtype) + (ea < -LINK_R).astype(x.dtype)
        h_in = jnp.concatenate([x[row], ea], axis=1)
        d_in = h_in.shape[1]
        d_pad = _ceil_to(d_in, 8)
        h_in = jnp.pad(h_in, ((0, 0), (0, d_pad - d_in)))

        layers1 = _pad_layers(lp["mlp1"], d_pad)
        lat1 = lp["mlp1"][-1][0].shape[0]
        h = _mlp_pallas(h_in, layers1, 1024)[:, :lat1]

        s = jax.ops.segment_sum(h, col, num_segments=n)
        cnt = jax.ops.segment_sum(jnp.ones((h.shape[0], 1), x.dtype), col,
                                  num_segments=n)
        mean = s / jnp.maximum(cnt, 1.0)
        mx = _segment_max0(h, col, n)

        h2_in = jnp.concatenate([x, mean, mx, s], axis=1)
        d2 = h2_in.shape[1]
        d2_pad = _ceil_to(d2, 8)
        h2_in = jnp.pad(h2_in, ((0, 0), (0, d2_pad - d2)))
        layers2 = _pad_layers(lp["mlp2"], d2_pad)
        lat2 = lp["mlp2"][-1][0].shape[0]
        h2 = _mlp_pallas(h2_in, layers2, 1000)[:, :lat2]

        x = jax.nn.relu(jnp.concatenate([x[:, :3], h2], axis=1))

    feats = x[:, 3:]
    addp = jax.ops.segment_sum(feats, batch, num_segments=n_graphs)
    cntg = jax.ops.segment_sum(jnp.ones((feats.shape[0], 1), x.dtype), batch,
                               num_segments=n_graphs)
    meanp = addp / jnp.maximum(cntg, 1.0)
    maxp = _segment_max0(feats, batch, n_graphs)
    pooled = jnp.concatenate([addp, meanp, maxp], axis=1)
    d3 = pooled.shape[1]
    d3_pad = _ceil_to(d3, 8)
    pooled = jnp.pad(pooled, ((0, 0), (0, d3_pad - d3)))
    layers3 = _pad_layers(params["lin"], d3_pad)
    n_out = params["lin"][-1][0].shape[0]
    out = _mlp_pallas(pooled, layers3, n_graphs)[:, :n_out]
    return out
